# baseline (speedup 1.0000x reference)
import functools

import jax
import jax.numpy as jnp
from jax import lax
from jax.experimental import pallas as pl
from jax.experimental.pallas import tpu as pltpu
from jax.experimental.pallas import tpu_sc as plsc

_B = 8
_H = 16
_P = 64
_D = 64
_NPOS = 6
_ROWS_PER_J = _P * _P
_NC = 2
_NS = 16
_NW = _NC * _NS
_QUARTERS = _NW // _B
_CHUNK = _ROWS_PER_J // _QUARTERS
_GATHER = 128
_ROUND = 512
_NROUND = _CHUNK // _ROUND
_RG = _ROUND // _GATHER
_L = 16
_REP = 32

_mesh = plsc.VectorSubcoreMesh(core_axis_name="c", subcore_axis_name="s")


@functools.partial(
    pl.kernel,
    mesh=_mesh,
    out_type=jax.ShapeDtypeStruct((_B * _ROWS_PER_J, 2 * _D), jnp.float32),
    scratch_types=[
        pltpu.VMEM((8, 128), jnp.int32),
        pltpu.VMEM((8, 128), jnp.int32),
        pltpu.VMEM((_ROUND, 2 * _D), jnp.float32),
        pltpu.SemaphoreType.DMA,
        pltpu.SemaphoreType.DMA,
    ],
)
def _encode(idx_hbm, tp_hbm, g_hbm, idx_v, sidx_v, rows_v, gsem, wsem):
    wid = lax.axis_index("s") * _NC + lax.axis_index("c")
    j = wid % _B
    q = wid // _B

    lane = lax.iota(jnp.int32, _L)

    row0 = pl.multiple_of(j * 32 + q * 8, 8)
    pltpu.sync_copy(idx_hbm.at[pl.ds(row0, 8)], idx_v)
    for k in range(8):
        for g in range(_GATHER // _L):
            rep = (g * _L) % _REP
            salt = _NPOS * (lane + rep)
            sidx_v[k, pl.ds(g * _L, _L)] = idx_v[k, pl.ds(g * _L, _L)] + salt

    write = None
    for r in range(_NROUND):
        if write is not None:
            write.wait()
        gathers = [
            pltpu.async_copy(
                tp_hbm.at[sidx_v.at[r * _RG + i]],
                rows_v.at[pl.ds(i * _GATHER, _GATHER)],
                gsem,
            )
            for i in range(_RG)
        ]
        for g in gathers:
            g.wait()
        base = pl.multiple_of(
            j * _ROWS_PER_J + q * _CHUNK + r * _ROUND, _ROUND
        )
        write = pltpu.async_copy(rows_v, g_hbm.at[pl.ds(base, _ROUND)], wsem)
    write.wait()


_PC = 4


def _bcast_body(g_ref, out_ref):
    g = g_ref[...]
    g4 = g[:, :, :_D].reshape(_B, _PC, _P, _D)
    out6 = jnp.broadcast_to(g4[None, None], (_B, 2, _B, _PC, _P, _D))
    out_ref[...] = out6.reshape(_B, _H, _PC, _P, _D)


_broadcast = pl.pallas_call(
    _bcast_body,
    grid=(_P // _PC,),
    in_specs=[pl.BlockSpec((_B, _PC * _P, 2 * _D), lambda c: (0, c, 0))],
    out_specs=pl.BlockSpec(
        (_B, _H, _PC, _P, _D), lambda c: (0, 0, c, 0, 0)
    ),
    out_shape=jax.ShapeDtypeStruct((_B, _H, _P, _P, _D), jnp.float32),
)


def kernel(src_seq, structure_emb):
    idx2d = src_seq.reshape(_B * _ROWS_PER_J // 128, 128).astype(jnp.int32)
    emb = structure_emb.astype(jnp.float32)
    tp = jnp.tile(jnp.pad(emb, ((0, 0), (0, 2 * _D - _D))), (_REP, 1))
    g = _encode(idx2d, tp)
    g3 = g.reshape(_B, _ROWS_PER_J, 2 * _D)
    return _broadcast(g3)


# speedup vs baseline: 1.0024x; 1.0024x over previous
import functools

import jax
import jax.numpy as jnp
from jax import lax
from jax.experimental import pallas as pl
from jax.experimental.pallas import tpu as pltpu
from jax.experimental.pallas import tpu_sc as plsc

_B = 8
_H = 16
_P = 64
_D = 64
_NPOS = 6
_ROWS_PER_J = _P * _P
_NC = 2
_NS = 16
_NW = _NC * _NS
_QUARTERS = _NW // _B
_CHUNK = _ROWS_PER_J // _QUARTERS
_GATHER = 128
_ROUND = 512
_NROUND = _CHUNK // _ROUND
_RG = _ROUND // _GATHER
_L = 16
_REP = 64

_mesh = plsc.VectorSubcoreMesh(core_axis_name="c", subcore_axis_name="s")


@functools.partial(
    pl.kernel,
    mesh=_mesh,
    out_type=jax.ShapeDtypeStruct((_B * _ROWS_PER_J, 2 * _D), jnp.float32),
    scratch_types=[
        pltpu.VMEM((8, 128), jnp.int32),
        pltpu.VMEM((8, 128), jnp.int32),
        pltpu.VMEM((_ROUND, 2 * _D), jnp.float32),
        pltpu.SemaphoreType.DMA,
        pltpu.SemaphoreType.DMA,
    ],
)
def _encode(idx_hbm, tp_hbm, g_hbm, idx_v, sidx_v, rows_v, gsem, wsem):
    wid = lax.axis_index("s") * _NC + lax.axis_index("c")
    j = wid % _B
    q = wid // _B

    lane = lax.iota(jnp.int32, _L)

    row0 = pl.multiple_of(j * 32 + q * 8, 8)
    pltpu.sync_copy(idx_hbm.at[pl.ds(row0, 8)], idx_v)
    for k in range(8):
        for g in range(_GATHER // _L):
            rep = (g * _L) % _REP
            salt = _NPOS * (lane + rep)
            sidx_v[k, pl.ds(g * _L, _L)] = idx_v[k, pl.ds(g * _L, _L)] + salt

    write = None
    for r in range(_NROUND):
        if write is not None:
            write.wait()
        gathers = [
            pltpu.async_copy(
                tp_hbm.at[sidx_v.at[r * _RG + i]],
                rows_v.at[pl.ds(i * _GATHER, _GATHER)],
                gsem,
            )
            for i in range(_RG)
        ]
        for g in gathers:
            g.wait()
        base = pl.multiple_of(
            j * _ROWS_PER_J + q * _CHUNK + r * _ROUND, _ROUND
        )
        write = pltpu.async_copy(rows_v, g_hbm.at[pl.ds(base, _ROUND)], wsem)
    write.wait()


_PC = 4


def _bcast_body(g_ref, out_ref):
    g = g_ref[...]
    g4 = g[:, :, :_D].reshape(_B, _PC, _P, _D)
    out6 = jnp.broadcast_to(g4[None, None], (_B, 2, _B, _PC, _P, _D))
    out_ref[...] = out6.reshape(_B, _H, _PC, _P, _D)


_broadcast = pl.pallas_call(
    _bcast_body,
    grid=(_P // _PC,),
    in_specs=[pl.BlockSpec((_B, _PC * _P, 2 * _D), lambda c: (0, c, 0))],
    out_specs=pl.BlockSpec(
        (_B, _H, _PC, _P, _D), lambda c: (0, 0, c, 0, 0)
    ),
    out_shape=jax.ShapeDtypeStruct((_B, _H, _P, _P, _D), jnp.float32),
)


def kernel(src_seq, structure_emb):
    idx2d = src_seq.reshape(_B * _ROWS_PER_J // 128, 128).astype(jnp.int32)
    emb = structure_emb.astype(jnp.float32)
    tp = jnp.tile(jnp.pad(emb, ((0, 0), (0, 2 * _D - _D))), (_REP, 1))
    g = _encode(idx2d, tp)
    g3 = g.reshape(_B, _ROWS_PER_J, 2 * _D)
    return _broadcast(g3)
